# baseline (speedup 1.0000x reference)
import jax
import jax.numpy as jnp
from jax import lax
from jax.experimental import pallas as pl
from jax.experimental.pallas import tpu as pltpu
from jax.experimental.pallas import tpu_sc as plsc

BATCH = 4096
BLOCK = 200
EMB = 64
NC = 2
NS = 16
NW = NC * NS

TOTAL = BATCH * BLOCK
PW = TOTAL // NW
SUB = 80
CH = 400
SPC = CH // SUB
NCHUNK = PW // CH
NSUB = PW // SUB


def _body(idx_hbm, tok_hbm, pos_hbm, out_hbm, idx_v, rows_v, gsem):
    wid = lax.axis_index("s") * NC + lax.axis_index("c")
    pltpu.sync_copy(idx_hbm.at[wid], idx_v)

    def chunk(g, carry):
        pltpu.sync_copy(pos_hbm, rows_v)
        for j in range(SPC):
            pltpu.async_copy(
                tok_hbm.at[idx_v.at[g * SPC + j]],
                rows_v.at[pl.ds(j * SUB, SUB)],
                gsem,
                add=True,
            ).wait()
        pltpu.sync_copy(rows_v, out_hbm.at[wid, g])
        return carry

    lax.fori_loop(0, NCHUNK, chunk, 0)


@jax.jit
def kernel(input, tok_table, pos_table):
    idx = input.reshape(NW, NSUB, SUB)
    pos_tiled = jnp.tile(pos_table, (CH // BLOCK, 1))
    mesh = plsc.VectorSubcoreMesh(core_axis_name="c", subcore_axis_name="s")
    out = pl.kernel(
        _body,
        out_type=jax.ShapeDtypeStruct((NW, NCHUNK, CH, EMB), jnp.float32),
        mesh=mesh,
        scratch_types=[
            pltpu.VMEM((NSUB, SUB), jnp.int32),
            pltpu.VMEM((CH, EMB), jnp.float32),
            pltpu.SemaphoreType.DMA,
        ],
        compiler_params=pltpu.CompilerParams(use_tc_tiling_on_sc=False),
    )(idx, tok_table, pos_tiled)
    return out.reshape(BATCH, BLOCK, EMB)


# speedup vs baseline: 1.0636x; 1.0636x over previous
import jax
import jax.numpy as jnp
from jax import lax
from jax.experimental import pallas as pl
from jax.experimental.pallas import tpu as pltpu
from jax.experimental.pallas import tpu_sc as plsc

BATCH = 4096
BLOCK = 200
EMB = 64
NC = 2
NS = 16
NW = NC * NS

TOTAL = BATCH * BLOCK
PW = TOTAL // NW
SUB = 80
CH = 400
SPC = CH // SUB
NCHUNK = PW // CH
NSUB = PW // SUB
NBUF = 4


def _body(idx_hbm, tok_hbm, pos_hbm, out_hbm, idx_v, rows_v, psem, gsem, osem):
    wid = lax.axis_index("s") * NC + lax.axis_index("c")
    pltpu.sync_copy(idx_hbm.at[wid], idx_v)

    def fire_init(g):
        s = lax.rem(g, NBUF) if not isinstance(g, int) else g % NBUF
        pltpu.async_copy(pos_hbm, rows_v.at[s], psem.at[s])

    def wait_init(g):
        s = lax.rem(g, NBUF) if not isinstance(g, int) else g % NBUF
        pltpu.make_async_copy(pos_hbm, rows_v.at[s], psem.at[s]).wait()

    def fire_gathers(g):
        s = lax.rem(g, NBUF) if not isinstance(g, int) else g % NBUF
        for j in range(SPC):
            pltpu.async_copy(
                tok_hbm.at[idx_v.at[g * SPC + j]],
                rows_v.at[s].at[pl.ds(j * SUB, SUB)],
                gsem.at[s],
                add=True,
            )

    def wait_gathers(g):
        s = lax.rem(g, NBUF) if not isinstance(g, int) else g % NBUF
        pltpu.make_async_copy(pos_hbm, rows_v.at[s], gsem.at[s]).wait()

    def fire_store(g):
        s = lax.rem(g, NBUF) if not isinstance(g, int) else g % NBUF
        pltpu.async_copy(rows_v.at[s], out_hbm.at[wid, g], osem.at[s])

    def wait_store(g):
        s = lax.rem(g, NBUF) if not isinstance(g, int) else g % NBUF
        pltpu.make_async_copy(rows_v.at[s], out_hbm.at[wid, 0], osem.at[s]).wait()

    fire_init(0)
    fire_init(1)
    wait_init(0)
    fire_gathers(0)
    fire_init(2)
    wait_init(1)
    fire_gathers(1)
    wait_gathers(0)
    fire_store(0)
    fire_init(3)

    def step(g, carry):
        wait_init(g)
        fire_gathers(g)
        wait_gathers(g - 1)
        fire_store(g - 1)
        wait_store(g - 2)
        fire_init(g + 2)
        return carry

    lax.fori_loop(2, NCHUNK - 2, step, 0)

    for g in (NCHUNK - 2, NCHUNK - 1):
        wait_init(g)
        fire_gathers(g)
        wait_gathers(g - 1)
        fire_store(g - 1)
        wait_store(g - 2)
    wait_gathers(NCHUNK - 1)
    fire_store(NCHUNK - 1)
    wait_store(NCHUNK - 2)
    wait_store(NCHUNK - 1)


@jax.jit
def kernel(input, tok_table, pos_table):
    idx = input.reshape(NW, NSUB, SUB)
    pos_tiled = jnp.tile(pos_table, (CH // BLOCK, 1))
    mesh = plsc.VectorSubcoreMesh(core_axis_name="c", subcore_axis_name="s")
    out = pl.kernel(
        _body,
        out_type=jax.ShapeDtypeStruct((NW, NCHUNK, CH, EMB), jnp.float32),
        mesh=mesh,
        scratch_types=[
            pltpu.VMEM((NSUB, SUB), jnp.int32),
            pltpu.VMEM((NBUF, CH, EMB), jnp.float32),
            pltpu.SemaphoreType.DMA((NBUF,)),
            pltpu.SemaphoreType.DMA((NBUF,)),
            pltpu.SemaphoreType.DMA((NBUF,)),
        ],
        compiler_params=pltpu.CompilerParams(use_tc_tiling_on_sc=False),
    )(idx, tok_table, pos_tiled)
    return out.reshape(BATCH, BLOCK, EMB)
